# baseline (speedup 1.0000x reference)
import functools

import jax
import jax.numpy as jnp
from jax.experimental import pallas as pl
from jax.experimental.pallas import tpu as pltpu

_MIN_CLIP = 1e-06
_NBUF = 4
_BW = 4096


def _loss_kernel(x_hbm, act_hbm, c_ref, o_ref,
                 x_buf, a_buf, x_sem, a_sem, nw: int, n_steps: int):
    d = c_ref.shape[0]

    def start_in(slot, step):
        n = step // nw
        w = (step % nw) * _BW
        pltpu.make_async_copy(
            x_hbm.at[n, :, pl.ds(w, _BW)], x_buf.at[slot], x_sem.at[slot]
        ).start()
        pltpu.make_async_copy(
            act_hbm.at[n, :, pl.ds(w, _BW)], a_buf.at[slot], a_sem.at[slot]
        ).start()

    def wait_in(slot):
        pltpu.make_async_copy(
            x_hbm.at[0, :, pl.ds(0, _BW)], x_buf.at[slot], x_sem.at[slot]
        ).wait()
        pltpu.make_async_copy(
            act_hbm.at[0, :, pl.ds(0, _BW)], a_buf.at[slot], a_sem.at[slot]
        ).wait()

    for s in range(_NBUF - 1):
        start_in(s, s)

    c = c_ref[...]
    c2 = jnp.sum(c * c, axis=0, keepdims=True)

    def body(step, acc):
        cur = jax.lax.rem(step, _NBUF)
        nxt = jax.lax.rem(step + _NBUF - 1, _NBUF)

        @pl.when(step + _NBUF - 1 < n_steps)
        def _():
            start_in(nxt, step + _NBUF - 1)

        wait_in(cur)
        x = x_buf[cur]
        a = a_buf[cur]
        x2 = jnp.sum(x * x, axis=0, keepdims=True)
        ones = jnp.ones_like(x2)
        xa = jnp.concatenate([x, x2, ones], axis=0)
        m = jax.lax.dot_general(
            xa, a, (((1,), (1,)), ((), ())),
            preferred_element_type=jnp.float32)
        part = (-2.0 * jnp.sum(c * m[:d])
                + jnp.sum(m[d:d + 1])
                + jnp.sum(c2 * m[d + 1:d + 2]))
        return acc + part

    loss = jax.lax.fori_loop(0, n_steps, body, jnp.float32(0.0))
    o_ref[0, 0] = jnp.maximum(loss, _MIN_CLIP)


@jax.jit
def kernel(x, c, act):
    n, d, wh = x.shape
    ch = c.shape[1]
    nw = wh // _BW
    n_steps = n * nw
    body = functools.partial(_loss_kernel, nw=nw, n_steps=n_steps)
    loss = pl.pallas_call(
        body,
        in_specs=[
            pl.BlockSpec(memory_space=pltpu.MemorySpace.HBM),
            pl.BlockSpec(memory_space=pltpu.MemorySpace.HBM),
            pl.BlockSpec(memory_space=pltpu.VMEM),
        ],
        out_specs=pl.BlockSpec(memory_space=pltpu.SMEM),
        out_shape=jax.ShapeDtypeStruct((1, 1), jnp.float32),
        scratch_shapes=[
            pltpu.VMEM((_NBUF, d, _BW), jnp.float32),
            pltpu.VMEM((_NBUF, ch, _BW), jnp.float32),
            pltpu.SemaphoreType.DMA((_NBUF,)),
            pltpu.SemaphoreType.DMA((_NBUF,)),
        ],
    )(x, act, c)
    return loss[0, 0]
